# Initial kernel scaffold; baseline (speedup 1.0000x reference)
import jax
import jax.numpy as jnp
from jax.experimental import pallas as pl


def kernel(inputs, state, support0_indices, support0_values, support1_indices, support1_values, weight, biases):
    raise NotImplementedError("write your pallas kernel here")


# speedup vs baseline: 1.7773x; 1.7773x over previous
import functools

import jax
import jax.numpy as jnp
from jax import lax
from jax.experimental import pallas as pl
from jax.experimental.pallas import tpu as pltpu
from jax.experimental.pallas import tpu_sc as plsc

N = 10000
B = 16
E = 160000
NTILE = 16
NCORE = 2
CHUNK = 128
EDGES_PER_TILE = 10240
NCHUNK = EDGES_PER_TILE // CHUNK
EP = EDGES_PER_TILE * NTILE
NP = 10240
ROWS_PER_TILE = NP // NTILE
RT = 1000
NT = N // RT


def _proj_body(inp_ref, st_ref, w_ref, b_ref, *out_refs):
    xs = [jnp.concatenate([inp_ref[g], st_ref[g]], axis=1) for g in range(4)]
    for m in range(5):
        cols = []
        for g in range(4):
            z = jnp.dot(xs[g], w_ref[m], preferred_element_type=jnp.float32)
            if m == 0:
                z = z + b_ref[0][None, :]
            cols.append(z)
        out_refs[m][0] = jnp.concatenate(cols, axis=1)


def _project(inputs, state, w5, bias2d):
    grid = (4, NT)
    zspec = pl.BlockSpec((1, RT, 128), lambda p, t: (p, t, 0))
    return pl.pallas_call(
        _proj_body,
        grid=grid,
        in_specs=[
            pl.BlockSpec((4, RT, 32), lambda p, t: (p, t, 0)),
            pl.BlockSpec((4, RT, 32), lambda p, t: (p, t, 0)),
            pl.BlockSpec((5, 64, 32), lambda p, t: (0, 0, 0)),
            pl.BlockSpec((1, 32), lambda p, t: (0, 0)),
        ],
        out_specs=[zspec] * 5,
        out_shape=[jax.ShapeDtypeStruct((4, NP, 128), jnp.float32)] * 5,
    )(inputs, state, w5, bias2d)


def _spmm_body(x_hbm, init_hbm, src_hbm, dst_hbm, val_hbm, out_hbm,
               acc_sh, src_v, dst_v, val_v, gbuf, gsem):
    c = lax.axis_index("c")
    s = lax.axis_index("s")

    def one_pass(j):
        p = c * 2 + j
        pltpu.sync_copy(init_hbm.at[p, pl.ds(s * ROWS_PER_TILE, ROWS_PER_TILE)],
                        acc_sh.at[pl.ds(s * ROWS_PER_TILE, ROWS_PER_TILE)])
        plsc.subcore_barrier()

        def chunk(i, carry):
            pltpu.sync_copy(src_hbm.at[p, s, pl.ds(i, 1)], src_v)
            pltpu.sync_copy(dst_hbm.at[s, pl.ds(i, 1)], dst_v)
            pltpu.async_copy(x_hbm.at[src_v.at[0]], gbuf, gsem).wait()
            pltpu.sync_copy(val_hbm.at[s, i], val_v)

            def scale(e, carry2):
                v = val_v[e]
                for q in range(8):
                    sl = pl.ds(q * 16, 16)
                    gbuf[e, sl] = gbuf[e, sl] * v
                return carry2

            lax.fori_loop(0, CHUNK, scale, 0, unroll=False)
            pltpu.sync_copy(gbuf, acc_sh.at[dst_v.at[0]], add=True)
            return carry

        lax.fori_loop(0, NCHUNK, chunk, 0, unroll=False)
        plsc.subcore_barrier()
        pltpu.sync_copy(acc_sh.at[pl.ds(s * ROWS_PER_TILE, ROWS_PER_TILE)],
                        out_hbm.at[p, pl.ds(s * ROWS_PER_TILE, ROWS_PER_TILE)])
        plsc.subcore_barrier()

    one_pass(0)
    one_pass(1)


def _make_spmm():
    mesh = plsc.VectorSubcoreMesh(core_axis_name="c", subcore_axis_name="s",
                                  num_cores=NCORE, num_subcores=NTILE)
    return pl.kernel(
        _spmm_body,
        out_type=jax.ShapeDtypeStruct((4, NP, 128), jnp.float32),
        mesh=mesh,
        scratch_types=[
            pltpu.MemorySpace.VMEM_SHARED((NP, 128), jnp.float32),
            pltpu.MemorySpace.VMEM((1, CHUNK), jnp.int32),
            pltpu.MemorySpace.VMEM((1, CHUNK), jnp.int32),
            pltpu.MemorySpace.VMEM((CHUNK, 16), jnp.float32),
            pltpu.MemorySpace.VMEM((CHUNK, 128), jnp.float32),
            pltpu.SemaphoreType.DMA,
        ],
    )


def kernel(inputs, state, support0_indices, support0_values,
           support1_indices, support1_values, weight, biases):
    w5 = weight.reshape(64, 5, 32).transpose(1, 0, 2)
    bias2d = biases.reshape(1, 32)
    z0, z1, z2, z3, z4 = _project(inputs, state, w5, bias2d)

    def prep(idx, vals):
        src = idx[1].astype(jnp.int32)
        dst = idx[0].astype(jnp.int32)
        pad = EP - E
        src = jnp.concatenate([src, jnp.zeros((pad,), jnp.int32)])
        dst = jnp.concatenate([dst, jnp.zeros((pad,), jnp.int32)])
        v = jnp.concatenate([vals, jnp.zeros((pad,), jnp.float32)])
        srcp = (src[None, :] + (jnp.arange(4, dtype=jnp.int32) * NP)[:, None])
        srcp = srcp.reshape(4, NTILE, NCHUNK, CHUNK)
        dstp = dst.reshape(NTILE, NCHUNK, CHUNK)
        v16 = jnp.broadcast_to(v[:, None], (EP, 16))
        v16 = v16.reshape(NTILE, NCHUNK, CHUNK, 16)
        return srcp, dstp, v16

    s0_src, s0_dst, s0_val = prep(support0_indices, support0_values)
    s1_src, s1_dst, s1_val = prep(support1_indices, support1_values)

    spmm = _make_spmm()

    def run(x, init, srcp, dstp, v16):
        return spmm(x.reshape(4 * NP, 128), init, srcp, dstp, v16)

    w0 = run(z2, z1, s0_src, s0_dst, s0_val)
    p = run(w0, z0, s0_src, s0_dst, s0_val)
    w1 = run(z4, z3, s1_src, s1_dst, s1_val)
    f = run(w1, p, s1_src, s1_dst, s1_val)

    out = f[:, :N, :].reshape(4, N, 4, 32).transpose(0, 2, 1, 3).reshape(B, N, 32)
    return out


# speedup vs baseline: 2.0454x; 1.1508x over previous
import functools

import jax
import jax.numpy as jnp
from jax import lax
from jax.experimental import pallas as pl
from jax.experimental.pallas import tpu as pltpu
from jax.experimental.pallas import tpu_sc as plsc

N = 10000
B = 16
E = 160000
NTILE = 16
NCORE = 2
CHUNK = 80
EDGES_PER_TILE = 10240
NCHUNK = EDGES_PER_TILE // CHUNK
EP = EDGES_PER_TILE * NTILE
NP = 10240
ROWS_PER_TILE = NP // NTILE
RT = 1000
NT = N // RT


def _proj_body(inp_ref, st_ref, w_ref, b_ref, *out_refs):
    xs = [jnp.concatenate([inp_ref[g], st_ref[g]], axis=1) for g in range(4)]
    for m in range(5):
        cols = []
        for g in range(4):
            z = jnp.dot(xs[g], w_ref[m], preferred_element_type=jnp.float32)
            if m == 0:
                z = z + b_ref[0][None, :]
            cols.append(z)
        out_refs[m][0] = jnp.concatenate(cols, axis=1)


def _project(inputs, state, w5, bias2d):
    grid = (4, NT)
    zspec = pl.BlockSpec((1, RT, 128), lambda p, t: (p, t, 0))
    return pl.pallas_call(
        _proj_body,
        grid=grid,
        in_specs=[
            pl.BlockSpec((4, RT, 32), lambda p, t: (p, t, 0)),
            pl.BlockSpec((4, RT, 32), lambda p, t: (p, t, 0)),
            pl.BlockSpec((5, 64, 32), lambda p, t: (0, 0, 0)),
            pl.BlockSpec((1, 32), lambda p, t: (0, 0)),
        ],
        out_specs=[zspec] * 5,
        out_shape=[jax.ShapeDtypeStruct((4, NP, 128), jnp.float32)] * 5,
    )(inputs, state, w5, bias2d)


def _spmm_body(x_hbm, init_hbm, sd_hbm, val_hbm, out_hbm,
               acc_sh, sd_a, sd_b, val_a, val_b, g_a, g_b,
               gs_a, gs_b, ss_a, ss_b):
    c = lax.axis_index("c")
    s = lax.axis_index("s")

    def one_pass(j):
        p = c * 2 + j
        pltpu.sync_copy(init_hbm.at[p, pl.ds(s * ROWS_PER_TILE, ROWS_PER_TILE)],
                        acc_sh.at[pl.ds(s * ROWS_PER_TILE, ROWS_PER_TILE)])
        plsc.subcore_barrier()

        def prefetch(i, sd_v, val_v, g_v, gsem):
            pltpu.sync_copy(sd_hbm.at[p, s, i], sd_v)
            pltpu.sync_copy(val_hbm.at[s, i], val_v)
            pltpu.async_copy(x_hbm.at[sd_v.at[0]], g_v, gsem)

        prefetch(0, sd_a, val_a, g_a, gs_a)
        prefetch(1, sd_b, val_b, g_b, gs_b)

        def process(sd_v, val_v, g_v, gsem, ssem):
            pltpu.make_async_copy(x_hbm.at[sd_v.at[0]], g_v, gsem).wait()

            def scale(e, c2):
                v = val_v[e]
                for q in range(8):
                    sl = pl.ds(q * 16, 16)
                    g_v[e, sl] = g_v[e, sl] * v
                return c2

            lax.fori_loop(0, CHUNK, scale, 0, unroll=2)
            return pltpu.async_copy(g_v, acc_sh.at[sd_v.at[1]], ssem, add=True)

        def pair(i2, carry):
            i = i2 * 2
            h_a = process(sd_a, val_a, g_a, gs_a, ss_a)
            h_b = process(sd_b, val_b, g_b, gs_b, ss_b)

            @pl.when(i + 2 < NCHUNK)
            def _():
                h_a.wait()
                prefetch(i + 2, sd_a, val_a, g_a, gs_a)

            @pl.when(i + 3 < NCHUNK)
            def _():
                h_b.wait()
                prefetch(i + 3, sd_b, val_b, g_b, gs_b)

            return carry

        lax.fori_loop(0, NCHUNK // 2, pair, 0, unroll=False)
        pltpu.make_async_copy(g_a, acc_sh.at[sd_a.at[1]], ss_a).wait()
        pltpu.make_async_copy(g_b, acc_sh.at[sd_b.at[1]], ss_b).wait()
        plsc.subcore_barrier()
        pltpu.sync_copy(acc_sh.at[pl.ds(s * ROWS_PER_TILE, ROWS_PER_TILE)],
                        out_hbm.at[p, pl.ds(s * ROWS_PER_TILE, ROWS_PER_TILE)])
        plsc.subcore_barrier()

    one_pass(0)
    one_pass(1)


def _make_spmm():
    mesh = plsc.VectorSubcoreMesh(core_axis_name="c", subcore_axis_name="s",
                                  num_cores=NCORE, num_subcores=NTILE)
    return pl.kernel(
        _spmm_body,
        out_type=jax.ShapeDtypeStruct((4, NP, 128), jnp.float32),
        mesh=mesh,
        scratch_types=[
            pltpu.MemorySpace.VMEM_SHARED((NP, 128), jnp.float32),
            pltpu.MemorySpace.VMEM((2, CHUNK), jnp.int32),
            pltpu.MemorySpace.VMEM((2, CHUNK), jnp.int32),
            pltpu.MemorySpace.VMEM((CHUNK, 16), jnp.float32),
            pltpu.MemorySpace.VMEM((CHUNK, 16), jnp.float32),
            pltpu.MemorySpace.VMEM((CHUNK, 128), jnp.float32),
            pltpu.MemorySpace.VMEM((CHUNK, 128), jnp.float32),
            pltpu.SemaphoreType.DMA,
            pltpu.SemaphoreType.DMA,
            pltpu.SemaphoreType.DMA,
            pltpu.SemaphoreType.DMA,
        ],
    )


def kernel(inputs, state, support0_indices, support0_values,
           support1_indices, support1_values, weight, biases):
    w5 = weight.reshape(64, 5, 32).transpose(1, 0, 2)
    bias2d = biases.reshape(1, 32)
    z0, z1, z2, z3, z4 = _project(inputs, state, w5, bias2d)

    def prep(idx, vals):
        src = idx[1].astype(jnp.int32)
        dst = idx[0].astype(jnp.int32)
        pad = EP - E
        src = jnp.concatenate([src, jnp.zeros((pad,), jnp.int32)])
        dst = jnp.concatenate([dst, jnp.zeros((pad,), jnp.int32)])
        v = jnp.concatenate([vals, jnp.zeros((pad,), jnp.float32)])
        srcp = (src[None, :] + (jnp.arange(4, dtype=jnp.int32) * NP)[:, None])
        srcp = srcp.reshape(4, NTILE, NCHUNK, CHUNK)
        dstp = jnp.broadcast_to(
            dst.reshape(1, NTILE, NCHUNK, CHUNK), (4, NTILE, NCHUNK, CHUNK))
        sd = jnp.stack([srcp, dstp], axis=3)
        v16 = jnp.broadcast_to(v[:, None], (EP, 16))
        v16 = v16.reshape(NTILE, NCHUNK, CHUNK, 16)
        return sd, v16

    sd0, v0 = prep(support0_indices, support0_values)
    sd1, v1 = prep(support1_indices, support1_values)

    spmm = _make_spmm()

    def run(x, init, sd, v16):
        return spmm(x.reshape(4 * NP, 128), init, sd, v16)

    w0 = run(z2, z1, sd0, v0)
    p = run(w0, z0, sd0, v0)
    w1 = run(z4, z3, sd1, v1)
    f = run(w1, p, sd1, v1)

    out = f[:, :N, :].reshape(4, N, 4, 32).transpose(0, 2, 1, 3).reshape(B, N, 32)
    return out


# speedup vs baseline: 2.1925x; 1.0719x over previous
import functools

import jax
import jax.numpy as jnp
from jax import lax
from jax.experimental import pallas as pl
from jax.experimental.pallas import tpu as pltpu
from jax.experimental.pallas import tpu_sc as plsc

N = 10000
B = 16
E = 160000
NTILE = 16
NCORE = 2
CHUNK = 128
EDGES_PER_TILE = 10240
NCHUNK = EDGES_PER_TILE // CHUNK
EP = EDGES_PER_TILE * NTILE
NP = 10112
ROWS_PER_TILE = NP // NTILE
RT = 1000
NT = N // RT


def _proj_body(inp_ref, st_ref, w_ref, b_ref, *out_refs):
    xs = [jnp.concatenate([inp_ref[g], st_ref[g]], axis=1) for g in range(4)]
    for m in range(5):
        cols = []
        for g in range(4):
            z = jnp.dot(xs[g], w_ref[m], preferred_element_type=jnp.float32)
            if m == 0:
                z = z + b_ref[0][None, :]
            cols.append(z)
        out_refs[m][0] = jnp.concatenate(cols, axis=1)


def _project(inputs, state, w5, bias2d):
    grid = (4, NT)
    zspec = pl.BlockSpec((1, RT, 128), lambda p, t: (p, t, 0))
    return pl.pallas_call(
        _proj_body,
        grid=grid,
        in_specs=[
            pl.BlockSpec((4, RT, 32), lambda p, t: (p, t, 0)),
            pl.BlockSpec((4, RT, 32), lambda p, t: (p, t, 0)),
            pl.BlockSpec((5, 64, 32), lambda p, t: (0, 0, 0)),
            pl.BlockSpec((1, 32), lambda p, t: (0, 0)),
        ],
        out_specs=[zspec] * 5,
        out_shape=[jax.ShapeDtypeStruct((4, NP, 128), jnp.float32)] * 5,
    )(inputs, state, w5, bias2d)


def _spmm_body(x_hbm, init_hbm, sd_hbm, val_hbm, out_hbm,
               acc_sh, sd_a, sd_b, val_v, g_a, g_b,
               gs_a, gs_b, ss_a, ss_b):
    c = lax.axis_index("c")
    s = lax.axis_index("s")

    def one_pass(j):
        p = c * 2 + j
        pltpu.sync_copy(init_hbm.at[p, pl.ds(s * ROWS_PER_TILE, ROWS_PER_TILE)],
                        acc_sh.at[pl.ds(s * ROWS_PER_TILE, ROWS_PER_TILE)])
        plsc.subcore_barrier()

        def prefetch(i, sd_v, g_v, gsem):
            pltpu.sync_copy(sd_hbm.at[p, s, i], sd_v)
            pltpu.async_copy(x_hbm.at[sd_v.at[0]], g_v, gsem)

        prefetch(0, sd_a, g_a, gs_a)
        prefetch(1, sd_b, g_b, gs_b)

        def process(i, sd_v, g_v, gsem, ssem):
            pltpu.sync_copy(val_hbm.at[s, i], val_v)
            pltpu.make_async_copy(x_hbm.at[sd_v.at[0]], g_v, gsem).wait()

            def scale(e, c2):
                v = val_v[e]
                for q in range(8):
                    sl = pl.ds(q * 16, 16)
                    g_v[e, sl] = g_v[e, sl] * v
                return c2

            lax.fori_loop(0, CHUNK, scale, 0, unroll=2)
            return pltpu.async_copy(g_v, acc_sh.at[sd_v.at[1]], ssem, add=True)

        def pair(i2, carry):
            i = i2 * 2
            h_a = process(i, sd_a, g_a, gs_a, ss_a)
            h_b = process(i + 1, sd_b, g_b, gs_b, ss_b)

            @pl.when(i + 2 < NCHUNK)
            def _():
                h_a.wait()
                prefetch(i + 2, sd_a, g_a, gs_a)

            @pl.when(i + 3 < NCHUNK)
            def _():
                h_b.wait()
                prefetch(i + 3, sd_b, g_b, gs_b)

            return carry

        lax.fori_loop(0, NCHUNK // 2, pair, 0, unroll=False)
        pltpu.make_async_copy(g_a, acc_sh.at[sd_a.at[1]], ss_a).wait()
        pltpu.make_async_copy(g_b, acc_sh.at[sd_b.at[1]], ss_b).wait()
        plsc.subcore_barrier()
        pltpu.sync_copy(acc_sh.at[pl.ds(s * ROWS_PER_TILE, ROWS_PER_TILE)],
                        out_hbm.at[p, pl.ds(s * ROWS_PER_TILE, ROWS_PER_TILE)])
        plsc.subcore_barrier()

    one_pass(0)
    one_pass(1)


def _make_spmm():
    mesh = plsc.VectorSubcoreMesh(core_axis_name="c", subcore_axis_name="s",
                                  num_cores=NCORE, num_subcores=NTILE)
    return pl.kernel(
        _spmm_body,
        out_type=jax.ShapeDtypeStruct((4, NP, 128), jnp.float32),
        mesh=mesh,
        scratch_types=[
            pltpu.MemorySpace.VMEM_SHARED((NP, 128), jnp.float32),
            pltpu.MemorySpace.VMEM((2, CHUNK), jnp.int32),
            pltpu.MemorySpace.VMEM((2, CHUNK), jnp.int32),
            pltpu.MemorySpace.VMEM((CHUNK, 16), jnp.float32),
            pltpu.MemorySpace.VMEM((CHUNK, 128), jnp.float32),
            pltpu.MemorySpace.VMEM((CHUNK, 128), jnp.float32),
            pltpu.SemaphoreType.DMA,
            pltpu.SemaphoreType.DMA,
            pltpu.SemaphoreType.DMA,
            pltpu.SemaphoreType.DMA,
        ],
    )


def kernel(inputs, state, support0_indices, support0_values,
           support1_indices, support1_values, weight, biases):
    w5 = weight.reshape(64, 5, 32).transpose(1, 0, 2)
    bias2d = biases.reshape(1, 32)
    z0, z1, z2, z3, z4 = _project(inputs, state, w5, bias2d)

    def prep(idx, vals):
        src = idx[1].astype(jnp.int32)
        dst = idx[0].astype(jnp.int32)
        pad = EP - E
        src = jnp.concatenate([src, jnp.zeros((pad,), jnp.int32)])
        dst = jnp.concatenate([dst, jnp.zeros((pad,), jnp.int32)])
        v = jnp.concatenate([vals, jnp.zeros((pad,), jnp.float32)])
        srcp = (src[None, :] + (jnp.arange(4, dtype=jnp.int32) * NP)[:, None])
        srcp = srcp.reshape(4, NTILE, NCHUNK, CHUNK)
        dstp = jnp.broadcast_to(
            dst.reshape(1, NTILE, NCHUNK, CHUNK), (4, NTILE, NCHUNK, CHUNK))
        sd = jnp.stack([srcp, dstp], axis=3)
        v16 = jnp.broadcast_to(v[:, None], (EP, 16))
        v16 = v16.reshape(NTILE, NCHUNK, CHUNK, 16)
        return sd, v16

    sd0, v0 = prep(support0_indices, support0_values)
    sd1, v1 = prep(support1_indices, support1_values)

    spmm = _make_spmm()

    def run(x, init, sd, v16):
        return spmm(x.reshape(4 * NP, 128), init, sd, v16)

    w0 = run(z2, z1, sd0, v0)
    p = run(w0, z0, sd0, v0)
    w1 = run(z4, z3, sd1, v1)
    f = run(w1, p, sd1, v1)

    out = f[:, :N, :].reshape(4, N, 4, 32).transpose(0, 2, 1, 3).reshape(B, N, 32)
    return out


# speedup vs baseline: 2.1997x; 1.0033x over previous
import functools

import jax
import jax.numpy as jnp
from jax import lax
from jax.experimental import pallas as pl
from jax.experimental.pallas import tpu as pltpu
from jax.experimental.pallas import tpu_sc as plsc

N = 10000
B = 16
E = 160000
NTILE = 16
NCORE = 2
CHUNK = 128
EDGES_PER_TILE = 10240
NCHUNK = EDGES_PER_TILE // CHUNK
EP = EDGES_PER_TILE * NTILE
NP = 10112
ROWS_PER_TILE = NP // NTILE
RT = 1000
NT = N // RT


def _proj_body(inp_ref, st_ref, w_ref, b_ref, *out_refs):
    xs = [jnp.concatenate([inp_ref[g], st_ref[g]], axis=1) for g in range(4)]
    for m in range(5):
        cols = []
        for g in range(4):
            z = jnp.dot(xs[g], w_ref[m], preferred_element_type=jnp.float32)
            if m == 0:
                z = z + b_ref[0][None, :]
            cols.append(z)
        out_refs[m][0] = jnp.concatenate(cols, axis=1)


def _project(inputs, state, w5, bias2d):
    grid = (4, NT)
    zspec = pl.BlockSpec((1, RT, 128), lambda p, t: (p, t, 0))
    return pl.pallas_call(
        _proj_body,
        grid=grid,
        in_specs=[
            pl.BlockSpec((4, RT, 32), lambda p, t: (p, t, 0)),
            pl.BlockSpec((4, RT, 32), lambda p, t: (p, t, 0)),
            pl.BlockSpec((5, 64, 32), lambda p, t: (0, 0, 0)),
            pl.BlockSpec((1, 32), lambda p, t: (0, 0)),
        ],
        out_specs=[zspec] * 5,
        out_shape=[jax.ShapeDtypeStruct((4, NP, 128), jnp.float32)] * 5,
    )(inputs, state, w5, bias2d)


def _spmm_body(x_hbm, init_hbm, sd_hbm, val_hbm, out_hbm,
               acc_sh, sd_a, sd_b, val_v, g_a, g_b,
               gs_a, gs_b, ss_a, ss_b, vs):
    c = lax.axis_index("c")
    s = lax.axis_index("s")

    def one_pass(j):
        p = c * 2 + j
        pltpu.sync_copy(init_hbm.at[p, pl.ds(s * ROWS_PER_TILE, ROWS_PER_TILE)],
                        acc_sh.at[pl.ds(s * ROWS_PER_TILE, ROWS_PER_TILE)])
        plsc.subcore_barrier()

        def prefetch(i, sd_v, g_v, gsem):
            pltpu.sync_copy(sd_hbm.at[p, s, i], sd_v)
            pltpu.async_copy(x_hbm.at[sd_v.at[0]], g_v, gsem)

        prefetch(0, sd_a, g_a, gs_a)
        prefetch(1, sd_b, g_b, gs_b)

        def process(i, sd_v, g_v, gsem, ssem):
            h_val = pltpu.async_copy(val_hbm.at[s, i], val_v, vs)
            pltpu.make_async_copy(x_hbm.at[sd_v.at[0]], g_v, gsem).wait()
            h_val.wait()

            def scale(e, c2):
                v = val_v[e]
                for q in range(8):
                    sl = pl.ds(q * 16, 16)
                    g_v[e, sl] = g_v[e, sl] * v
                return c2

            lax.fori_loop(0, CHUNK, scale, 0, unroll=2)
            return pltpu.async_copy(g_v, acc_sh.at[sd_v.at[1]], ssem, add=True)

        def pair(i2, carry):
            i = i2 * 2
            h_a = process(i, sd_a, g_a, gs_a, ss_a)
            h_b = process(i + 1, sd_b, g_b, gs_b, ss_b)

            @pl.when(i + 2 < NCHUNK)
            def _():
                h_a.wait()
                prefetch(i + 2, sd_a, g_a, gs_a)

            @pl.when(i + 3 < NCHUNK)
            def _():
                h_b.wait()
                prefetch(i + 3, sd_b, g_b, gs_b)

            return carry

        lax.fori_loop(0, NCHUNK // 2, pair, 0, unroll=False)
        pltpu.make_async_copy(g_a, acc_sh.at[sd_a.at[1]], ss_a).wait()
        pltpu.make_async_copy(g_b, acc_sh.at[sd_b.at[1]], ss_b).wait()
        plsc.subcore_barrier()
        pltpu.sync_copy(acc_sh.at[pl.ds(s * ROWS_PER_TILE, ROWS_PER_TILE)],
                        out_hbm.at[p, pl.ds(s * ROWS_PER_TILE, ROWS_PER_TILE)])
        plsc.subcore_barrier()

    one_pass(0)
    one_pass(1)


def _make_spmm():
    mesh = plsc.VectorSubcoreMesh(core_axis_name="c", subcore_axis_name="s",
                                  num_cores=NCORE, num_subcores=NTILE)
    return pl.kernel(
        _spmm_body,
        out_type=jax.ShapeDtypeStruct((4, NP, 128), jnp.float32),
        mesh=mesh,
        scratch_types=[
            pltpu.MemorySpace.VMEM_SHARED((NP, 128), jnp.float32),
            pltpu.MemorySpace.VMEM((2, CHUNK), jnp.int32),
            pltpu.MemorySpace.VMEM((2, CHUNK), jnp.int32),
            pltpu.MemorySpace.VMEM((CHUNK, 16), jnp.float32),
            pltpu.MemorySpace.VMEM((CHUNK, 128), jnp.float32),
            pltpu.MemorySpace.VMEM((CHUNK, 128), jnp.float32),
            pltpu.SemaphoreType.DMA,
            pltpu.SemaphoreType.DMA,
            pltpu.SemaphoreType.DMA,
            pltpu.SemaphoreType.DMA,
            pltpu.SemaphoreType.DMA,
        ],
    )


def kernel(inputs, state, support0_indices, support0_values,
           support1_indices, support1_values, weight, biases):
    w5 = weight.reshape(64, 5, 32).transpose(1, 0, 2)
    bias2d = biases.reshape(1, 32)
    z0, z1, z2, z3, z4 = _project(inputs, state, w5, bias2d)

    def prep(idx, vals):
        src = idx[1].astype(jnp.int32)
        dst = idx[0].astype(jnp.int32)
        pad = EP - E
        src = jnp.concatenate([src, jnp.zeros((pad,), jnp.int32)])
        dst = jnp.concatenate([dst, jnp.zeros((pad,), jnp.int32)])
        v = jnp.concatenate([vals, jnp.zeros((pad,), jnp.float32)])
        srcp = (src[None, :] + (jnp.arange(4, dtype=jnp.int32) * NP)[:, None])
        srcp = srcp.reshape(4, NTILE, NCHUNK, CHUNK)
        dstp = jnp.broadcast_to(
            dst.reshape(1, NTILE, NCHUNK, CHUNK), (4, NTILE, NCHUNK, CHUNK))
        sd = jnp.stack([srcp, dstp], axis=3)
        v16 = jnp.broadcast_to(v[:, None], (EP, 16))
        v16 = v16.reshape(NTILE, NCHUNK, CHUNK, 16)
        return sd, v16

    sd0, v0 = prep(support0_indices, support0_values)
    sd1, v1 = prep(support1_indices, support1_values)

    spmm = _make_spmm()

    def run(x, init, sd, v16):
        return spmm(x.reshape(4 * NP, 128), init, sd, v16)

    w0 = run(z2, z1, sd0, v0)
    p = run(w0, z0, sd0, v0)
    w1 = run(z4, z3, sd1, v1)
    f = run(w1, p, sd1, v1)

    out = f[:, :N, :].reshape(4, N, 4, 32).transpose(0, 2, 1, 3).reshape(B, N, 32)
    return out
